# baseline (speedup 1.0000x reference)
import functools

import jax
import jax.numpy as jnp
from jax import lax
from jax.experimental import pallas as pl
from jax.experimental.pallas import tpu as pltpu
from jax.experimental.pallas import tpu_sc as plsc

_INFO = plsc.get_sparse_core_info()
_NC = _INFO.num_cores
_NS = _INFO.num_subcores
_NW = _NC * _NS

_CHUNK = 128


@functools.lru_cache(maxsize=None)
def _make_gather(V, D, B):
    assert B % _NW == 0
    b_per_w = B // _NW
    nchunk = b_per_w // _CHUNK
    mesh = plsc.VectorSubcoreMesh(core_axis_name="c", subcore_axis_name="s")

    @functools.partial(
        pl.kernel,
        mesh=mesh,
        out_type=jax.ShapeDtypeStruct((_NW, D, b_per_w), jnp.float32),
        scratch_types=[
            pltpu.VMEM((b_per_w,), jnp.int32),
            pltpu.VMEM((nchunk, _CHUNK), jnp.int32),
            pltpu.VMEM((D, b_per_w), jnp.float32),
            pltpu.SemaphoreType.DMA,
        ],
    )
    def gather(flat_hbm, idx_hbm, out_hbm, idx_v, list_v, out_v, sem):
        w = lax.axis_index("s") * _NC + lax.axis_index("c")
        base = w * b_per_w
        pltpu.sync_copy(idx_hbm.at[pl.ds(base, b_per_w)], idx_v)

        def body(f, carry):
            fbase = f * V
            for c in range(nchunk):
                for h in range(_CHUNK // 16):
                    sl = pl.ds(c * _CHUNK + h * 16, 16)
                    list_v[c, pl.ds(h * 16, 16)] = idx_v[sl] + fbase
            copies = [
                pltpu.async_copy(
                    flat_hbm.at[list_v.at[c]],
                    out_v.at[f, pl.ds(c * _CHUNK, _CHUNK)],
                    sem,
                )
                for c in range(nchunk)
            ]
            for cp in copies:
                cp.wait()
            return carry

        lax.fori_loop(0, D, body, 0)
        pltpu.sync_copy(out_v, out_hbm.at[w])

    return gather


def kernel(input, indices, prestride, poststride, output_elements):
    n = indices.shape[0]
    d = input.shape[-1]
    v = input.shape[0]
    flat = input.T.reshape(d * v)
    out_w = _make_gather(v, d, n)(flat, indices)
    out_t = out_w.transpose(1, 0, 2).reshape(d, n)
    return out_t.T.reshape(1, n, d)
